# Initial kernel scaffold; baseline (speedup 1.0000x reference)
import jax
import jax.numpy as jnp
from jax.experimental import pallas as pl


def kernel(node_states, edge_src_0, edge_dst_0, edge_src_1, edge_dst_1, node_to_graph_idx, ref_root_ids, ref_root_graph_idx, edge_feat_0, edge_feat_1, W0, W1, W_self, b):
    raise NotImplementedError("write your pallas kernel here")


# speedup vs baseline: 3.0503x; 3.0503x over previous
import jax
import jax.numpy as jnp
from jax import lax
from jax.experimental import pallas as pl
from jax.experimental.pallas import tpu as pltpu
from jax.experimental.pallas import tpu_sc as plsc

N = 10000
D = 128
H = 16
E = 160000

NC = 2
NS = 16
NW = NC * NS
CH = 128
NCHUNKS = E // CH
BASE_CHUNKS = NCHUNKS // NW
HP = 32
NPAD = 10240
ROWS_PER_TILE = NPAD // NS
KCH = ROWS_PER_TILE // CH


def _dense_pre_body(ns_ref, w_ref, b_ref, p0_ref, p1_ref, s_ref):
    res = jnp.dot(ns_ref[...], w_ref[...], preferred_element_type=jnp.float32)
    p0_ref[...] = res[:, :D]
    p1_ref[...] = res[:, D:2 * D]
    s_ref[...] = res[:, 2 * D:] + b_ref[...]


def _dense_pre(ns, wcat, b):
    blk = 1000
    return pl.pallas_call(
        _dense_pre_body,
        grid=(N // blk,),
        in_specs=[
            pl.BlockSpec((blk, D), lambda i: (i, 0)),
            pl.BlockSpec((D, 3 * D), lambda i: (0, 0)),
            pl.BlockSpec((1, D), lambda i: (0, 0)),
        ],
        out_specs=[
            pl.BlockSpec((blk, D), lambda i: (i, 0)),
            pl.BlockSpec((blk, D), lambda i: (i, 0)),
            pl.BlockSpec((blk, D), lambda i: (i, 0)),
        ],
        out_shape=[
            jax.ShapeDtypeStruct((N, D), jnp.float32),
            jax.ShapeDtypeStruct((N, D), jnp.float32),
            jax.ShapeDtypeStruct((N, D), jnp.float32),
        ],
    )(ns, wcat, b)


def _p_body(p0, p1, src0, dst0, src1, dst1, zp,
            out_p,
            src_v, dst_v, rows_v, acc_p, sem):
    cid = lax.axis_index("c")
    sid = lax.axis_index("s")
    wid = sid * NC + cid
    row0 = sid * ROWS_PER_TILE

    pltpu.sync_copy(zp, rows_v)
    for k in range(KCH):
        pltpu.sync_copy(rows_v, acc_p.at[pl.ds(row0 + k * CH, CH), :])
    plsc.subcore_barrier()

    def make_body(p, src, dst):
        def body(r, carry):
            c = r * NW + wid

            @pl.when(c < NCHUNKS)
            def _():
                base = c * CH
                pltpu.sync_copy(src.at[pl.ds(base, CH)], src_v)
                pltpu.sync_copy(dst.at[pl.ds(base, CH)], dst_v)
                pltpu.async_copy(p.at[src_v], rows_v, sem).wait()
                pltpu.sync_copy(rows_v, acc_p.at[dst_v], add=True)

            return carry
        return body

    lax.fori_loop(0, BASE_CHUNKS + 1, make_body(p0, src0, dst0), 0)
    lax.fori_loop(0, BASE_CHUNKS + 1, make_body(p1, src1, dst1), 0)
    plsc.subcore_barrier()

    orow = cid * NPAD + row0
    for k in range(KCH):
        pltpu.sync_copy(acc_p.at[pl.ds(row0 + k * CH, CH), :], rows_v)
        pltpu.sync_copy(rows_v, out_p.at[pl.ds(orow + k * CH, CH), :])


def _p_pass(p0, p1, src0, dst0, src1, dst1, zp):
    mesh = plsc.VectorSubcoreMesh(core_axis_name="c", subcore_axis_name="s",
                                  num_cores=NC, num_subcores=NS)
    out = pl.kernel(
        _p_body,
        out_type=jax.ShapeDtypeStruct((NC * NPAD, D), jnp.float32),
        mesh=mesh,
        scratch_types=[
            pltpu.VMEM((CH,), jnp.int32),
            pltpu.VMEM((CH,), jnp.int32),
            pltpu.VMEM((CH, D), jnp.float32),
            pltpu.VMEM_SHARED((NPAD, D), jnp.float32),
            pltpu.SemaphoreType.DMA,
        ],
    )(p0, p1, src0, dst0, src1, dst1, zp)
    return out.reshape(NC, NPAD, D)


def _f_body(ef0, dst0, ef1, dst1, zp,
            out_f,
            dst_v, fbuf, acc_f, sem):
    cid = lax.axis_index("c")
    sid = lax.axis_index("s")
    wid = sid * NC + cid
    row0 = sid * ROWS_PER_TILE

    pltpu.sync_copy(zp, fbuf)
    for k in range(KCH):
        pltpu.sync_copy(fbuf, acc_f.at[pl.ds(row0 + k * CH, CH), :])
    plsc.subcore_barrier()

    def make_body(ef, dst):
        def body(r, carry):
            c = r * NW + wid

            @pl.when(c < NCHUNKS)
            def _():
                base = c * CH
                pltpu.sync_copy(dst.at[pl.ds(base, CH)], dst_v)
                pltpu.sync_copy(ef.at[pl.ds(base, CH), :], fbuf)
                pltpu.sync_copy(fbuf, acc_f.at[dst_v], add=True)

            return carry
        return body

    lax.fori_loop(0, BASE_CHUNKS + 1, make_body(ef0, dst0), 0)
    lax.fori_loop(0, BASE_CHUNKS + 1, make_body(ef1, dst1), 0)
    plsc.subcore_barrier()

    orow = cid * NPAD + row0
    for k in range(KCH):
        pltpu.sync_copy(acc_f.at[pl.ds(row0 + k * CH, CH), :], fbuf)
        pltpu.sync_copy(fbuf, out_f.at[pl.ds(orow + k * CH, CH), :])


def _f_pass(ef0, dst0, ef1, dst1, zp):
    mesh = plsc.VectorSubcoreMesh(core_axis_name="c", subcore_axis_name="s",
                                  num_cores=NC, num_subcores=NS)
    out_f = pl.kernel(
        _f_body,
        out_type=jax.ShapeDtypeStruct((NC * NPAD, D), jnp.float32),
        mesh=mesh,
        scratch_types=[
            pltpu.VMEM((CH,), jnp.int32),
            pltpu.VMEM((CH, D), jnp.float32),
            pltpu.VMEM_SHARED((NPAD, D), jnp.float32),
            pltpu.SemaphoreType.DMA,
        ],
    )(ef0, dst0, ef1, dst1, zp)
    return out_f.reshape(NC, NPAD, D)


def _final_body(s_ref, pp_ref, ff_ref, wf_ref, out_ref):
    acc = s_ref[...] + pp_ref[0] + pp_ref[1]
    acc = acc + jnp.dot(ff_ref[0] + ff_ref[1], wf_ref[...],
                        preferred_element_type=jnp.float32)
    out_ref[...] = jnp.maximum(acc, 0.0)


def _final(s, pp, ff, wf):
    blk = 1000
    return pl.pallas_call(
        _final_body,
        grid=(N // blk,),
        in_specs=[
            pl.BlockSpec((blk, D), lambda i: (i, 0)),
            pl.BlockSpec((NC, blk, D), lambda i: (0, i, 0)),
            pl.BlockSpec((NC, blk, D), lambda i: (0, i, 0)),
            pl.BlockSpec((D, D), lambda i: (0, 0)),
        ],
        out_specs=pl.BlockSpec((blk, D), lambda i: (i, 0)),
        out_shape=jax.ShapeDtypeStruct((N, D), jnp.float32),
    )(s, pp, ff, wf)


def kernel(node_states, edge_src_0, edge_dst_0, edge_src_1, edge_dst_1,
           node_to_graph_idx, ref_root_ids, ref_root_graph_idx,
           edge_feat_0, edge_feat_1, W0, W1, W_self, b):
    ns = node_states.astype(jnp.float32)
    wcat = jnp.concatenate([W0[:D], W1[:D], W_self], axis=1)
    p0, p1, s = _dense_pre(ns, wcat, b.reshape(1, D))

    src0 = edge_src_0.astype(jnp.int32)
    dst0 = edge_dst_0.astype(jnp.int32)
    src1 = edge_src_1.astype(jnp.int32)
    dst1 = edge_dst_1.astype(jnp.int32)
    ef0p = jnp.pad(edge_feat_0.astype(jnp.float32), ((0, 0), (0, D - H)))
    ef1p = jnp.pad(edge_feat_1.astype(jnp.float32), ((0, 0), (H, D - 2 * H)))
    wf = jnp.concatenate([W0[D:], W1[D:], jnp.zeros((D - 2 * H, D), jnp.float32)])
    zp = jnp.zeros((CH, D), jnp.float32)

    pp = _p_pass(p0, p1, src0, dst0, src1, dst1, zp)
    ff = _f_pass(ef0p, dst0, ef1p, dst1, zp)

    return _final(s, pp, ff, wf)


# speedup vs baseline: 3.4535x; 1.1322x over previous
import jax
import jax.numpy as jnp
from jax import lax
from jax.experimental import pallas as pl
from jax.experimental.pallas import tpu as pltpu
from jax.experimental.pallas import tpu_sc as plsc

N = 10000
D = 128
H = 16
E = 160000

NC = 2
NS = 16
NW = NC * NS
CH = 128
NCHUNKS = E // CH
BASE_CHUNKS = NCHUNKS // NW
HP = 32
NPAD = 10240
ROWS_PER_TILE = NPAD // NS
KCH = ROWS_PER_TILE // CH


def _dense_pre_body(ns_ref, w_ref, b_ref, p0_ref, p1_ref, s_ref):
    res = jnp.dot(ns_ref[...], w_ref[...], preferred_element_type=jnp.float32)
    p0_ref[...] = res[:, :D]
    p1_ref[...] = res[:, D:2 * D]
    s_ref[...] = res[:, 2 * D:] + b_ref[...]


def _dense_pre(ns, wcat, b):
    blk = 1000
    return pl.pallas_call(
        _dense_pre_body,
        grid=(N // blk,),
        in_specs=[
            pl.BlockSpec((blk, D), lambda i: (i, 0)),
            pl.BlockSpec((D, 3 * D), lambda i: (0, 0)),
            pl.BlockSpec((1, D), lambda i: (0, 0)),
        ],
        out_specs=[
            pl.BlockSpec((blk, D), lambda i: (i, 0)),
            pl.BlockSpec((blk, D), lambda i: (i, 0)),
            pl.BlockSpec((blk, D), lambda i: (i, 0)),
        ],
        out_shape=[
            jax.ShapeDtypeStruct((N, D), jnp.float32),
            jax.ShapeDtypeStruct((N, D), jnp.float32),
            jax.ShapeDtypeStruct((N, D), jnp.float32),
        ],
    )(ns, wcat, b)


def _p_body(p0, p1, src0, dst0, src1, dst1, zp,
            out_p,
            src_v, dst_v, rows_v, src_w, dst_w, rows_w, acc_p, sem, sem2):
    cid = lax.axis_index("c")
    sid = lax.axis_index("s")
    wid = sid * NC + cid
    row0 = sid * ROWS_PER_TILE

    pltpu.sync_copy(zp, rows_v)
    for k in range(KCH):
        pltpu.sync_copy(rows_v, acc_p.at[pl.ds(row0 + k * CH, CH), :])
    plsc.subcore_barrier()

    def make_body(p, src, dst):
        def body(r, carry):
            ca = (2 * r) * NW + wid
            cb = (2 * r + 1) * NW + wid

            @pl.when(ca < NCHUNKS)
            def _():
                base = ca * CH
                pltpu.sync_copy(src.at[pl.ds(base, CH)], src_v)
                pltpu.sync_copy(dst.at[pl.ds(base, CH)], dst_v)
                ga = pltpu.async_copy(p.at[src_v], rows_v, sem)

                @pl.when(cb < NCHUNKS)
                def _():
                    base_b = cb * CH
                    pltpu.sync_copy(src.at[pl.ds(base_b, CH)], src_w)
                    pltpu.sync_copy(dst.at[pl.ds(base_b, CH)], dst_w)
                    gb = pltpu.async_copy(p.at[src_w], rows_w, sem2)
                    ga.wait()
                    pltpu.sync_copy(rows_v, acc_p.at[dst_v], add=True)
                    gb.wait()
                    pltpu.sync_copy(rows_w, acc_p.at[dst_w], add=True)

                @pl.when(cb >= NCHUNKS)
                def _():
                    ga.wait()
                    pltpu.sync_copy(rows_v, acc_p.at[dst_v], add=True)

            return carry
        return body

    lax.fori_loop(0, (BASE_CHUNKS + 2) // 2, make_body(p0, src0, dst0), 0)
    lax.fori_loop(0, (BASE_CHUNKS + 2) // 2, make_body(p1, src1, dst1), 0)
    plsc.subcore_barrier()

    orow = cid * NPAD + row0
    for k in range(KCH):
        pltpu.sync_copy(acc_p.at[pl.ds(row0 + k * CH, CH), :], rows_v)
        pltpu.sync_copy(rows_v, out_p.at[pl.ds(orow + k * CH, CH), :])


def _p_pass(p0, p1, src0, dst0, src1, dst1, zp):
    mesh = plsc.VectorSubcoreMesh(core_axis_name="c", subcore_axis_name="s",
                                  num_cores=NC, num_subcores=NS)
    out = pl.kernel(
        _p_body,
        out_type=jax.ShapeDtypeStruct((NC * NPAD, D), jnp.float32),
        mesh=mesh,
        scratch_types=[
            pltpu.VMEM((CH,), jnp.int32),
            pltpu.VMEM((CH,), jnp.int32),
            pltpu.VMEM((CH, D), jnp.float32),
            pltpu.VMEM((CH,), jnp.int32),
            pltpu.VMEM((CH,), jnp.int32),
            pltpu.VMEM((CH, D), jnp.float32),
            pltpu.VMEM_SHARED((NPAD, D), jnp.float32),
            pltpu.SemaphoreType.DMA,
            pltpu.SemaphoreType.DMA,
        ],
    )(p0, p1, src0, dst0, src1, dst1, zp)
    return out.reshape(NC, NPAD, D)


def _f_body(ef0, dst0, ef1, dst1, zp,
            out_f,
            dst_v, fbuf, acc_f, sem):
    cid = lax.axis_index("c")
    sid = lax.axis_index("s")
    wid = sid * NC + cid
    row0 = sid * ROWS_PER_TILE

    pltpu.sync_copy(zp, fbuf)
    for k in range(KCH):
        pltpu.sync_copy(fbuf, acc_f.at[pl.ds(row0 + k * CH, CH), :])
    plsc.subcore_barrier()

    def make_body(ef, dst):
        def body(r, carry):
            c = r * NW + wid

            @pl.when(c < NCHUNKS)
            def _():
                base = c * CH
                pltpu.sync_copy(dst.at[pl.ds(base, CH)], dst_v)
                pltpu.sync_copy(ef.at[pl.ds(base, CH), :], fbuf)
                pltpu.sync_copy(fbuf, acc_f.at[dst_v], add=True)

            return carry
        return body

    lax.fori_loop(0, BASE_CHUNKS + 1, make_body(ef0, dst0), 0)
    lax.fori_loop(0, BASE_CHUNKS + 1, make_body(ef1, dst1), 0)
    plsc.subcore_barrier()

    orow = cid * NPAD + row0
    for k in range(KCH):
        pltpu.sync_copy(acc_f.at[pl.ds(row0 + k * CH, CH), :], fbuf)
        pltpu.sync_copy(fbuf, out_f.at[pl.ds(orow + k * CH, CH), :])


def _f_pass(ef0, dst0, ef1, dst1, zp):
    mesh = plsc.VectorSubcoreMesh(core_axis_name="c", subcore_axis_name="s",
                                  num_cores=NC, num_subcores=NS)
    out_f = pl.kernel(
        _f_body,
        out_type=jax.ShapeDtypeStruct((NC * NPAD, D), jnp.float32),
        mesh=mesh,
        scratch_types=[
            pltpu.VMEM((CH,), jnp.int32),
            pltpu.VMEM((CH, D), jnp.float32),
            pltpu.VMEM_SHARED((NPAD, D), jnp.float32),
            pltpu.SemaphoreType.DMA,
        ],
    )(ef0, dst0, ef1, dst1, zp)
    return out_f.reshape(NC, NPAD, D)


def _final_body(s_ref, pp_ref, ff_ref, wf_ref, out_ref):
    acc = s_ref[...] + pp_ref[0] + pp_ref[1]
    acc = acc + jnp.dot(ff_ref[0] + ff_ref[1], wf_ref[...],
                        preferred_element_type=jnp.float32)
    out_ref[...] = jnp.maximum(acc, 0.0)


def _final(s, pp, ff, wf):
    blk = 1000
    return pl.pallas_call(
        _final_body,
        grid=(N // blk,),
        in_specs=[
            pl.BlockSpec((blk, D), lambda i: (i, 0)),
            pl.BlockSpec((NC, blk, D), lambda i: (0, i, 0)),
            pl.BlockSpec((NC, blk, D), lambda i: (0, i, 0)),
            pl.BlockSpec((D, D), lambda i: (0, 0)),
        ],
        out_specs=pl.BlockSpec((blk, D), lambda i: (i, 0)),
        out_shape=jax.ShapeDtypeStruct((N, D), jnp.float32),
    )(s, pp, ff, wf)


def kernel(node_states, edge_src_0, edge_dst_0, edge_src_1, edge_dst_1,
           node_to_graph_idx, ref_root_ids, ref_root_graph_idx,
           edge_feat_0, edge_feat_1, W0, W1, W_self, b):
    ns = node_states.astype(jnp.float32)
    wcat = jnp.concatenate([W0[:D], W1[:D], W_self], axis=1)
    p0, p1, s = _dense_pre(ns, wcat, b.reshape(1, D))

    src0 = edge_src_0.astype(jnp.int32)
    dst0 = edge_dst_0.astype(jnp.int32)
    src1 = edge_src_1.astype(jnp.int32)
    dst1 = edge_dst_1.astype(jnp.int32)
    ef0p = jnp.pad(edge_feat_0.astype(jnp.float32), ((0, 0), (0, D - H)))
    ef1p = jnp.pad(edge_feat_1.astype(jnp.float32), ((0, 0), (H, D - 2 * H)))
    wf = jnp.concatenate([W0[D:], W1[D:], jnp.zeros((D - 2 * H, D), jnp.float32)])
    zp = jnp.zeros((CH, D), jnp.float32)

    pp = _p_pass(p0, p1, src0, dst0, src1, dst1, zp)
    ff = _f_pass(ef0p, dst0, ef1p, dst1, zp)

    return _final(s, pp, ff, wf)


# speedup vs baseline: 3.8726x; 1.1214x over previous
import jax
import jax.numpy as jnp
from jax import lax
from jax.experimental import pallas as pl
from jax.experimental.pallas import tpu as pltpu
from jax.experimental.pallas import tpu_sc as plsc

N = 10000
D = 128
H = 16
E = 160000

NC = 2
NS = 16
NW = NC * NS
CH = 128
NCHUNKS = E // CH
BASE_CHUNKS = NCHUNKS // NW
HP = 32
NPAD = 10240
ROWS_PER_TILE = NPAD // NS
KCH = ROWS_PER_TILE // CH


def _dense_pre_body(ns_ref, w_ref, b_ref, p0_ref, p1_ref, s_ref):
    res = jnp.dot(ns_ref[...], w_ref[...], preferred_element_type=jnp.float32)
    p0_ref[...] = res[:, :D]
    p1_ref[...] = res[:, D:2 * D]
    s_ref[...] = res[:, 2 * D:] + b_ref[...]


def _dense_pre(ns, wcat, b):
    blk = 1000
    return pl.pallas_call(
        _dense_pre_body,
        grid=(N // blk,),
        in_specs=[
            pl.BlockSpec((blk, D), lambda i: (i, 0)),
            pl.BlockSpec((D, 3 * D), lambda i: (0, 0)),
            pl.BlockSpec((1, D), lambda i: (0, 0)),
        ],
        out_specs=[
            pl.BlockSpec((blk, D), lambda i: (i, 0)),
            pl.BlockSpec((blk, D), lambda i: (i, 0)),
            pl.BlockSpec((blk, D), lambda i: (i, 0)),
        ],
        out_shape=[
            jax.ShapeDtypeStruct((N, D), jnp.float32),
            jax.ShapeDtypeStruct((N, D), jnp.float32),
            jax.ShapeDtypeStruct((N, D), jnp.float32),
        ],
    )(ns, wcat, b)


def _p_body(p0, p1, src0, dst0, src1, dst1, zp,
            out_p,
            src_v, dst_v, rows_v, src_w, dst_w, rows_w, acc_p, sem, sem2):
    cid = lax.axis_index("c")
    sid = lax.axis_index("s")
    wid = sid * NC + cid
    row0 = sid * ROWS_PER_TILE

    pltpu.sync_copy(zp, rows_v)
    for k in range(KCH):
        pltpu.sync_copy(rows_v, acc_p.at[pl.ds(row0 + k * CH, CH), :])
    plsc.subcore_barrier()

    def make_body(p, src, dst):
        def body(r, carry):
            ca = (2 * r) * NW + wid
            cb = (2 * r + 1) * NW + wid

            @pl.when(ca < NCHUNKS)
            def _():
                base = ca * CH
                pltpu.sync_copy(src.at[pl.ds(base, CH)], src_v)
                pltpu.sync_copy(dst.at[pl.ds(base, CH)], dst_v)
                ga = pltpu.async_copy(p.at[src_v], rows_v, sem)

                @pl.when(cb < NCHUNKS)
                def _():
                    base_b = cb * CH
                    pltpu.sync_copy(src.at[pl.ds(base_b, CH)], src_w)
                    pltpu.sync_copy(dst.at[pl.ds(base_b, CH)], dst_w)
                    gb = pltpu.async_copy(p.at[src_w], rows_w, sem2)
                    ga.wait()
                    pltpu.sync_copy(rows_v, acc_p.at[dst_v], add=True)
                    gb.wait()
                    pltpu.sync_copy(rows_w, acc_p.at[dst_w], add=True)

                @pl.when(cb >= NCHUNKS)
                def _():
                    ga.wait()
                    pltpu.sync_copy(rows_v, acc_p.at[dst_v], add=True)

            return carry
        return body

    lax.fori_loop(0, (BASE_CHUNKS + 2) // 2, make_body(p0, src0, dst0), 0)
    lax.fori_loop(0, (BASE_CHUNKS + 2) // 2, make_body(p1, src1, dst1), 0)
    plsc.subcore_barrier()

    orow = cid * NPAD + row0
    for k in range(KCH):
        pltpu.sync_copy(acc_p.at[pl.ds(row0 + k * CH, CH), :], rows_v)
        pltpu.sync_copy(rows_v, out_p.at[pl.ds(orow + k * CH, CH), :])


def _p_pass(p0, p1, src0, dst0, src1, dst1, zp):
    mesh = plsc.VectorSubcoreMesh(core_axis_name="c", subcore_axis_name="s",
                                  num_cores=NC, num_subcores=NS)
    out = pl.kernel(
        _p_body,
        out_type=jax.ShapeDtypeStruct((NC * NPAD, D), jnp.float32),
        mesh=mesh,
        scratch_types=[
            pltpu.VMEM((CH,), jnp.int32),
            pltpu.VMEM((CH,), jnp.int32),
            pltpu.VMEM((CH, D), jnp.float32),
            pltpu.VMEM((CH,), jnp.int32),
            pltpu.VMEM((CH,), jnp.int32),
            pltpu.VMEM((CH, D), jnp.float32),
            pltpu.VMEM_SHARED((NPAD, D), jnp.float32),
            pltpu.SemaphoreType.DMA,
            pltpu.SemaphoreType.DMA,
        ],
    )(p0, p1, src0, dst0, src1, dst1, zp)
    return out.reshape(NC, NPAD, D)


def _f_body(ef0, dst0, ef1, dst1, zp,
            out_f,
            dst_v, fbuf, dst_w, fbuf2, acc_f, sem, sem2):
    cid = lax.axis_index("c")
    sid = lax.axis_index("s")
    wid = sid * NC + cid
    row0 = sid * ROWS_PER_TILE

    pltpu.sync_copy(zp, fbuf)
    for k in range(KCH):
        pltpu.sync_copy(fbuf, acc_f.at[pl.ds(row0 + k * CH, CH), :])
    plsc.subcore_barrier()

    def make_body(ef, dst):
        def body(r, carry):
            ca = (2 * r) * NW + wid
            cb = (2 * r + 1) * NW + wid

            @pl.when(ca < NCHUNKS)
            def _():
                base = ca * CH
                pltpu.sync_copy(dst.at[pl.ds(base, CH)], dst_v)
                ga = pltpu.async_copy(ef.at[pl.ds(base, CH), :], fbuf, sem)

                @pl.when(cb < NCHUNKS)
                def _():
                    base_b = cb * CH
                    pltpu.sync_copy(dst.at[pl.ds(base_b, CH)], dst_w)
                    gb = pltpu.async_copy(ef.at[pl.ds(base_b, CH), :], fbuf2, sem2)
                    ga.wait()
                    pltpu.sync_copy(fbuf, acc_f.at[dst_v], add=True)
                    gb.wait()
                    pltpu.sync_copy(fbuf2, acc_f.at[dst_w], add=True)

                @pl.when(cb >= NCHUNKS)
                def _():
                    ga.wait()
                    pltpu.sync_copy(fbuf, acc_f.at[dst_v], add=True)

            return carry
        return body

    lax.fori_loop(0, (BASE_CHUNKS + 2) // 2, make_body(ef0, dst0), 0)
    lax.fori_loop(0, (BASE_CHUNKS + 2) // 2, make_body(ef1, dst1), 0)
    plsc.subcore_barrier()

    orow = cid * NPAD + row0
    for k in range(KCH):
        pltpu.sync_copy(acc_f.at[pl.ds(row0 + k * CH, CH), :], fbuf)
        pltpu.sync_copy(fbuf, out_f.at[pl.ds(orow + k * CH, CH), :])


def _f_pass(ef0, dst0, ef1, dst1, zp):
    mesh = plsc.VectorSubcoreMesh(core_axis_name="c", subcore_axis_name="s",
                                  num_cores=NC, num_subcores=NS)
    out_f = pl.kernel(
        _f_body,
        out_type=jax.ShapeDtypeStruct((NC * NPAD, D), jnp.float32),
        mesh=mesh,
        scratch_types=[
            pltpu.VMEM((CH,), jnp.int32),
            pltpu.VMEM((CH, D), jnp.float32),
            pltpu.VMEM((CH,), jnp.int32),
            pltpu.VMEM((CH, D), jnp.float32),
            pltpu.VMEM_SHARED((NPAD, D), jnp.float32),
            pltpu.SemaphoreType.DMA,
            pltpu.SemaphoreType.DMA,
        ],
    )(ef0, dst0, ef1, dst1, zp)
    return out_f.reshape(NC, NPAD, D)


def _final_body(s_ref, pp_ref, ff_ref, wf_ref, out_ref):
    acc = s_ref[...] + pp_ref[0] + pp_ref[1]
    acc = acc + jnp.dot(ff_ref[0] + ff_ref[1], wf_ref[...],
                        preferred_element_type=jnp.float32)
    out_ref[...] = jnp.maximum(acc, 0.0)


def _final(s, pp, ff, wf):
    blk = 1000
    return pl.pallas_call(
        _final_body,
        grid=(N // blk,),
        in_specs=[
            pl.BlockSpec((blk, D), lambda i: (i, 0)),
            pl.BlockSpec((NC, blk, D), lambda i: (0, i, 0)),
            pl.BlockSpec((NC, blk, D), lambda i: (0, i, 0)),
            pl.BlockSpec((D, D), lambda i: (0, 0)),
        ],
        out_specs=pl.BlockSpec((blk, D), lambda i: (i, 0)),
        out_shape=jax.ShapeDtypeStruct((N, D), jnp.float32),
    )(s, pp, ff, wf)


def kernel(node_states, edge_src_0, edge_dst_0, edge_src_1, edge_dst_1,
           node_to_graph_idx, ref_root_ids, ref_root_graph_idx,
           edge_feat_0, edge_feat_1, W0, W1, W_self, b):
    ns = node_states.astype(jnp.float32)
    wcat = jnp.concatenate([W0[:D], W1[:D], W_self], axis=1)
    p0, p1, s = _dense_pre(ns, wcat, b.reshape(1, D))

    src0 = edge_src_0.astype(jnp.int32)
    dst0 = edge_dst_0.astype(jnp.int32)
    src1 = edge_src_1.astype(jnp.int32)
    dst1 = edge_dst_1.astype(jnp.int32)
    ef0p = jnp.pad(edge_feat_0.astype(jnp.float32), ((0, 0), (0, D - H)))
    ef1p = jnp.pad(edge_feat_1.astype(jnp.float32), ((0, 0), (H, D - 2 * H)))
    wf = jnp.concatenate([W0[D:], W1[D:], jnp.zeros((D - 2 * H, D), jnp.float32)])
    zp = jnp.zeros((CH, D), jnp.float32)

    pp = _p_pass(p0, p1, src0, dst0, src1, dst1, zp)
    ff = _f_pass(ef0p, dst0, ef1p, dst1, zp)

    return _final(s, pp, ff, wf)
